# Initial kernel scaffold; baseline (speedup 1.0000x reference)
import jax
import jax.numpy as jnp
from jax.experimental import pallas as pl


def kernel(features, W_proj, b_proj, ln_gamma, ln_beta, codebooks, residual_scales, temperature):
    raise NotImplementedError("write your pallas kernel here")


# speedup vs baseline: 3.0634x; 3.0634x over previous
import jax
import jax.numpy as jnp
from jax.experimental import pallas as pl
from jax.experimental.pallas import tpu as pltpu

_B = 4096
_D = 256
_K = 1024
_L = 4
_BB = 256


def _body(scal_ref, feat_ref, w_ref, bias_ref, g_ref, beta_ref, cb_ref,
          logits_ref, qsum_ref):
    s0 = scal_ref[0, 0]
    inv_t = scal_ref[0, 1]

    f = feat_ref[...]
    w = w_ref[...]
    h = jax.lax.dot_general(f, w, (((1,), (1,)), ((), ())),
                            preferred_element_type=jnp.float32)
    h = h + bias_ref[...]

    mu = jnp.mean(h, axis=-1, keepdims=True)
    var = jnp.mean((h - mu) * (h - mu), axis=-1, keepdims=True)
    h = (h - mu) * jax.lax.rsqrt(var + 1e-5)
    h = h * g_ref[...] + beta_ref[...]
    h = jnp.maximum(h, 0.0)

    rs = h * s0

    cb = cb_ref[...]
    cbn = jnp.sum(cb * cb, axis=-1)

    rown = jnp.sum(rs * rs, axis=-1, keepdims=True)
    cb0 = cb[0]
    cross = jax.lax.dot_general(rs, cb0, (((1,), (1,)), ((), ())),
                                preferred_element_type=jnp.float32)
    dist0 = rown + cbn[0][None, :] - 2.0 * cross
    logits_ref[:, 0:_K] = -dist0 * inv_t

    for lvl in range(1, _L):
        row = (-cbn[lvl] * inv_t)[None, :]
        logits_ref[:, lvl * _K:(lvl + 1) * _K] = jnp.broadcast_to(
            row, (_BB, _K))

    mu2 = jnp.mean(rs, axis=-1, keepdims=True)
    var2 = jnp.mean((rs - mu2) * (rs - mu2), axis=-1, keepdims=True)
    qsum_ref[...] = (rs - mu2) * jax.lax.rsqrt(var2 + 1e-5)


def kernel(features, W_proj, b_proj, ln_gamma, ln_beta, codebooks,
           residual_scales, temperature):
    inv_t = 1.0 / jnp.maximum(temperature, 0.01)
    scal = jnp.stack([residual_scales[0].astype(jnp.float32),
                      inv_t.astype(jnp.float32)]).reshape(1, 2)

    grid = (_B // _BB,)
    logits2d, qsum = pl.pallas_call(
        _body,
        grid=grid,
        in_specs=[
            pl.BlockSpec(memory_space=pltpu.SMEM),
            pl.BlockSpec((_BB, _D), lambda i: (i, 0)),
            pl.BlockSpec((_D, _D), lambda i: (0, 0)),
            pl.BlockSpec((1, _D), lambda i: (0, 0)),
            pl.BlockSpec((1, _D), lambda i: (0, 0)),
            pl.BlockSpec((1, _D), lambda i: (0, 0)),
            pl.BlockSpec((_L, _K, _D), lambda i: (0, 0, 0)),
        ],
        out_specs=[
            pl.BlockSpec((_BB, _L * _K), lambda i: (i, 0)),
            pl.BlockSpec((_BB, _D), lambda i: (i, 0)),
        ],
        out_shape=[
            jax.ShapeDtypeStruct((_B, _L * _K), jnp.float32),
            jax.ShapeDtypeStruct((_B, _D), jnp.float32),
        ],
        compiler_params=pltpu.CompilerParams(
            dimension_semantics=("arbitrary",)),
    )(
        scal,
        features,
        W_proj,
        b_proj.reshape(1, _D),
        ln_gamma.reshape(1, _D),
        ln_beta.reshape(1, _D),
        codebooks,
    )
    return logits2d.reshape(_B, _L, _K), qsum


# speedup vs baseline: 3.1014x; 1.0124x over previous
import jax
import jax.numpy as jnp
from jax.experimental import pallas as pl
from jax.experimental.pallas import tpu as pltpu

_B = 4096
_D = 256
_K = 1024
_L = 4
_BB = 256


def _body(scal_ref, feat_ref, w_ref, bias_ref, g_ref, beta_ref, cb_ref,
          logits_ref, qsum_ref, nrow_ref):
    s0 = scal_ref[0, 0]
    inv_t = scal_ref[0, 1]

    @pl.when(pl.program_id(0) == 0)
    def _():
        cb = cb_ref[...]
        nrow_ref[...] = jnp.sum(cb * cb, axis=-1) * (-inv_t)

    f = feat_ref[...]
    w = w_ref[...]
    h = jax.lax.dot_general(f, w, (((1,), (1,)), ((), ())),
                            preferred_element_type=jnp.float32)
    h = h + bias_ref[...]

    mu = jnp.mean(h, axis=-1, keepdims=True)
    var = jnp.mean((h - mu) * (h - mu), axis=-1, keepdims=True)
    h = (h - mu) * jax.lax.rsqrt(var + 1e-5)
    h = h * g_ref[...] + beta_ref[...]
    h = jnp.maximum(h, 0.0)

    rs = h * s0

    rown = jnp.sum(rs * rs, axis=-1, keepdims=True)
    cb0 = cb_ref[0]
    cross = jax.lax.dot_general(rs, cb0, (((1,), (1,)), ((), ())),
                                preferred_element_type=jnp.float32)
    logits_ref[:, 0:_K] = ((2.0 * inv_t) * cross - inv_t * rown
                           + nrow_ref[0][None, :])

    for lvl in range(1, _L):
        logits_ref[:, lvl * _K:(lvl + 1) * _K] = jnp.broadcast_to(
            nrow_ref[lvl][None, :], (_BB, _K))

    mu2 = jnp.mean(rs, axis=-1, keepdims=True)
    var2 = jnp.mean((rs - mu2) * (rs - mu2), axis=-1, keepdims=True)
    qsum_ref[...] = (rs - mu2) * jax.lax.rsqrt(var2 + 1e-5)


def kernel(features, W_proj, b_proj, ln_gamma, ln_beta, codebooks,
           residual_scales, temperature):
    inv_t = 1.0 / jnp.maximum(temperature, 0.01)
    scal = jnp.stack([residual_scales[0].astype(jnp.float32),
                      inv_t.astype(jnp.float32)]).reshape(1, 2)

    grid = (_B // _BB,)
    logits2d, qsum = pl.pallas_call(
        _body,
        grid=grid,
        in_specs=[
            pl.BlockSpec(memory_space=pltpu.SMEM),
            pl.BlockSpec((_BB, _D), lambda i: (i, 0)),
            pl.BlockSpec((_D, _D), lambda i: (0, 0)),
            pl.BlockSpec((1, _D), lambda i: (0, 0)),
            pl.BlockSpec((1, _D), lambda i: (0, 0)),
            pl.BlockSpec((1, _D), lambda i: (0, 0)),
            pl.BlockSpec((_L, _K, _D), lambda i: (0, 0, 0)),
        ],
        out_specs=[
            pl.BlockSpec((_BB, _L * _K), lambda i: (i, 0)),
            pl.BlockSpec((_BB, _D), lambda i: (i, 0)),
        ],
        out_shape=[
            jax.ShapeDtypeStruct((_B, _L * _K), jnp.float32),
            jax.ShapeDtypeStruct((_B, _D), jnp.float32),
        ],
        scratch_shapes=[pltpu.VMEM((_L, _K), jnp.float32)],
        compiler_params=pltpu.CompilerParams(
            dimension_semantics=("arbitrary",)),
    )(
        scal,
        features,
        W_proj,
        b_proj.reshape(1, _D),
        ln_gamma.reshape(1, _D),
        ln_beta.reshape(1, _D),
        codebooks,
    )
    return logits2d.reshape(_B, _L, _K), qsum
